# Initial kernel scaffold; baseline (speedup 1.0000x reference)
import jax
import jax.numpy as jnp
from jax.experimental import pallas as pl


def kernel(x, token_table, pos_table):
    raise NotImplementedError("write your pallas kernel here")


# speedup vs baseline: 2.1357x; 2.1357x over previous
import functools
import jax
import jax.numpy as jnp
from jax import lax
from jax.experimental import pallas as pl
from jax.experimental.pallas import tpu as pltpu
from jax.experimental.pallas import tpu_sc as plsc

MAXLEN = 200
EMBED = 64
LANES = 16
NC, NS = 2, 16
NW = NC * NS


@functools.partial(jax.jit, static_argnums=(0,))
def _run(B, xf, token_table, pos_table):
    BPW = B // NW
    CH = 256
    NCH = BPW // CH
    GS = 128

    mesh = plsc.VectorSubcoreMesh(core_axis_name="c", subcore_axis_name="s")

    @functools.partial(
        pl.kernel,
        out_type=jax.ShapeDtypeStruct((B, EMBED), jnp.float32),
        mesh=mesh,
        scratch_types=[
            pltpu.VMEM((BPW,), jnp.int32),
            pltpu.VMEM((MAXLEN, EMBED), jnp.float32),
            pltpu.VMEM((CH, EMBED), jnp.float32),
            pltpu.SemaphoreType.DMA,
        ],
        compiler_params=pltpu.CompilerParams(use_tc_tiling_on_sc=False),
    )
    def k(idx_hbm, tok_hbm, pos_hbm, out_hbm, idx_v, pos_v, buf, sem):
        wid = lax.axis_index("s") * NC + lax.axis_index("c")
        base = wid * BPW
        pltpu.sync_copy(idx_hbm.at[pl.ds(base, BPW)], idx_v)
        pltpu.sync_copy(pos_hbm, pos_v)

        def chunk_body(c, carry):
            cb = pl.multiple_of(c * CH, CH)
            cps = [
                pltpu.async_copy(
                    tok_hbm.at[idx_v.at[pl.ds(cb + g * GS, GS)]],
                    buf.at[pl.ds(g * GS, GS)],
                    sem,
                )
                for g in range(CH // GS)
            ]
            for cp in cps:
                cp.wait()

            def row_body(r, rcarry):
                p = lax.rem(cb + r, MAXLEN)
                for cc in range(EMBED // LANES):
                    sl = pl.ds(cc * LANES, LANES)
                    buf[r, sl] = buf[r, sl] + pos_v[p, sl]
                return rcarry

            lax.fori_loop(0, CH, row_body, 0)
            pltpu.sync_copy(buf, out_hbm.at[pl.ds(base + cb, CH)])
            return carry

        lax.fori_loop(0, NCH, chunk_body, 0)

    return k(xf, token_table, pos_table)


def kernel(x, token_table, pos_table):
    bt, t = x.shape
    B = bt * t
    xf = x.reshape(B).astype(jnp.int32)
    out = _run(B, xf, token_table, pos_table)
    return out.reshape(bt, t, EMBED)


# speedup vs baseline: 3.2489x; 1.5212x over previous
import functools
import jax
import jax.numpy as jnp
from jax import lax
from jax.experimental import pallas as pl
from jax.experimental.pallas import tpu as pltpu
from jax.experimental.pallas import tpu_sc as plsc

MAXLEN = 200
EMBED = 64
LANES = 16
NC, NS = 2, 16
NW = NC * NS
IW = 100
SUB = MAXLEN // IW
NBUF = 4
PF = 2


@functools.partial(jax.jit, static_argnums=(0,))
def _run(B, xf2d, token_table, pos_table):
    BPW = B // NW
    CH = MAXLEN
    NCH = BPW // CH
    assert NCH % NBUF == 0

    mesh = plsc.VectorSubcoreMesh(core_axis_name="c", subcore_axis_name="s")

    scratch = [
        pltpu.VMEM((NCH * SUB, IW), jnp.int32),
        pltpu.VMEM((MAXLEN, EMBED), jnp.float32),
    ]
    scratch += [pltpu.VMEM((CH, EMBED), jnp.float32) for _ in range(NBUF)]
    scratch += [pltpu.SemaphoreType.DMA for _ in range(2 * NBUF)]

    @functools.partial(
        pl.kernel,
        out_type=jax.ShapeDtypeStruct((B, EMBED), jnp.float32),
        mesh=mesh,
        scratch_types=scratch,
        compiler_params=pltpu.CompilerParams(use_tc_tiling_on_sc=False),
    )
    def k(idx_hbm, tok_hbm, pos_hbm, out_hbm, idx_v, pos_v, *rest):
        bufs = rest[:NBUF]
        gsems = rest[NBUF:2 * NBUF]
        ssems = rest[2 * NBUF:]

        wid = lax.axis_index("s") * NC + lax.axis_index("c")
        rowbase = wid * BPW
        pltpu.sync_copy(idx_hbm.at[pl.ds(wid * NCH * SUB, NCH * SUB)], idx_v)
        pltpu.sync_copy(pos_hbm, pos_v)

        def fire_gather(c, b):
            for s in range(SUB):
                pltpu.async_copy(
                    tok_hbm.at[idx_v.at[c * SUB + s]],
                    bufs[b].at[pl.ds(s * IW, IW)],
                    gsems[b],
                )

        def wait_gather(c, b):
            for s in range(SUB):
                pltpu.make_async_copy(
                    tok_hbm.at[idx_v.at[c * SUB + s]],
                    bufs[b].at[pl.ds(s * IW, IW)],
                    gsems[b],
                ).wait()

        def drain_scatter(b):
            pltpu.make_async_copy(
                bufs[b], out_hbm.at[pl.ds(0, CH)], ssems[b]
            ).wait()

        for b in range(PF):
            fire_gather(b, b)

        def outer(o, carry):
            for b in range(NBUF):
                i = o * NBUF + b
                j = i + PF
                jb = (b + PF) % NBUF

                @pl.when(j < NCH)
                def _prefetch():
                    @pl.when(j >= NBUF)
                    def _drain():
                        drain_scatter(jb)

                    fire_gather(j, jb)

                wait_gather(i, b)

                buf = bufs[b]

                @plsc.parallel_loop(0, CH, unroll=2)
                def _add(r):
                    for cc in range(EMBED // LANES):
                        sl = pl.ds(cc * LANES, LANES)
                        buf[r, sl] = buf[r, sl] + pos_v[r, sl]

                pltpu.async_copy(
                    buf, out_hbm.at[pl.ds(rowbase + i * CH, CH)], ssems[b]
                )
            return carry

        lax.fori_loop(0, NCH // NBUF, outer, 0)

        for b in range(NBUF):
            drain_scatter(b)

    return k(xf2d, token_table, pos_table)


def kernel(x, token_table, pos_table):
    bt, t = x.shape
    B = bt * t
    xf2d = x.reshape(B // IW, IW).astype(jnp.int32)
    out = _run(B, xf2d, token_table, pos_table)
    return out.reshape(bt, t, EMBED)
